# Initial kernel scaffold; baseline (speedup 1.0000x reference)
import functools

import jax
import jax.numpy as jnp
from jax import lax
from jax.experimental import pallas as pl
from jax.experimental.pallas import tpu as pltpu
from jax.experimental.pallas import tpu_sc as plsc

N = 10000
H = 256
O = 128
NLAYERS = 4
HALF = 5000
ROWS_SC = 5008
ROWS_TILE = 313
TRASH = 5000
EB = 128
NB = 158
NBP = 160
EMB_B = 3
NE_PAD = 32 * EMB_B * EB

_MESH = plsc.VectorSubcoreMesh(core_axis_name="c", subcore_axis_name="s")


def _embed_body(embed_hbm, hb_hbm, out_hbm, idx_v, rows_v, sem):
    c = lax.axis_index("c")
    s = lax.axis_index("s")
    wid = s * 2 + c
    pltpu.sync_copy(hb_hbm.at[wid], idx_v)
    for j in range(EMB_B):
        pltpu.async_copy(embed_hbm.at[idx_v.at[j]], rows_v, sem).wait()
        pltpu.sync_copy(rows_v, out_hbm.at[pl.ds(wid * (EMB_B * EB) + j * EB, EB)])


_embed_call = functools.partial(
    pl.kernel,
    mesh=_MESH,
    out_type=jax.ShapeDtypeStruct((NE_PAD, H), jnp.float32),
    scratch_types=[
        pltpu.VMEM((EMB_B, EB), jnp.int32),
        pltpu.VMEM((EB, H), jnp.float32),
        pltpu.SemaphoreType.DMA,
    ],
)(_embed_body)


def _agg_body(x_hbm, srcb_hbm, dstb_hbm, zeros_hbm, out_hbm,
              src_v, dst_v, rows0, rows1, accum, sem0, sem1):
    c = lax.axis_index("c")
    s = lax.axis_index("s")
    pltpu.sync_copy(srcb_hbm.at[s], src_v)
    pltpu.sync_copy(dstb_hbm.at[c, s], dst_v)
    pltpu.sync_copy(zeros_hbm.at[pl.ds(s * ROWS_TILE, ROWS_TILE)],
                    accum.at[pl.ds(s * ROWS_TILE, ROWS_TILE)])
    plsc.subcore_barrier()
    pltpu.async_copy(x_hbm.at[src_v.at[0]], rows0, sem0)
    pltpu.async_copy(x_hbm.at[src_v.at[1]], rows1, sem1)

    def body(k, carry):
        j0 = 2 * k
        j1 = j0 + 1
        pltpu.make_async_copy(x_hbm.at[src_v.at[j0]], rows0, sem0).wait()
        pltpu.sync_copy(rows0, accum.at[dst_v.at[j0]], add=True)
        pltpu.async_copy(x_hbm.at[src_v.at[j0 + 2]], rows0, sem0)
        pltpu.make_async_copy(x_hbm.at[src_v.at[j1]], rows1, sem1).wait()
        pltpu.sync_copy(rows1, accum.at[dst_v.at[j1]], add=True)
        pltpu.async_copy(x_hbm.at[src_v.at[j1 + 2]], rows1, sem1)
        return carry

    lax.fori_loop(0, NB // 2, body, 0)
    pltpu.make_async_copy(x_hbm.at[src_v.at[NB]], rows0, sem0).wait()
    pltpu.make_async_copy(x_hbm.at[src_v.at[NB + 1]], rows1, sem1).wait()
    plsc.subcore_barrier()
    pltpu.sync_copy(accum.at[pl.ds(s * ROWS_TILE, ROWS_TILE)],
                    out_hbm.at[c, pl.ds(s * ROWS_TILE, ROWS_TILE)])


_agg_call = functools.partial(
    pl.kernel,
    mesh=_MESH,
    out_type=jax.ShapeDtypeStruct((2, ROWS_SC, H), jnp.float32),
    scratch_types=[
        pltpu.VMEM((NBP, EB), jnp.int32),
        pltpu.VMEM((NBP, EB), jnp.int32),
        pltpu.VMEM((EB, H), jnp.float32),
        pltpu.VMEM((EB, H), jnp.float32),
        pltpu.VMEM_SHARED((ROWS_SC, H), jnp.float32),
        pltpu.SemaphoreType.DMA,
        pltpu.SemaphoreType.DMA,
    ],
)(_agg_body)


def _mlp_body(x_ref, agg_ref, w1_ref, w2_ref, gm_ref, bm_ref, go_ref, bo_ref,
              xo_ref, p_ref):
    x = x_ref[:N, :]
    agg = jnp.concatenate([agg_ref[0, :HALF, :], agg_ref[1, :HALF, :]], axis=0)
    z = 2.0 * x + agg
    z = jnp.dot(z, w1_ref[...], preferred_element_type=jnp.float32)
    m = jnp.mean(z, axis=0, keepdims=True)
    d = z - m
    var = jnp.mean(d * d, axis=0, keepdims=True)
    z = d * lax.rsqrt(var + 1e-5) * gm_ref[...] + bm_ref[...]
    z = jnp.maximum(z, 0.0)
    z = jnp.dot(z, w2_ref[...], preferred_element_type=jnp.float32)
    m2 = jnp.mean(z, axis=0, keepdims=True)
    d2 = z - m2
    var2 = jnp.mean(d2 * d2, axis=0, keepdims=True)
    z = d2 * lax.rsqrt(var2 + 1e-5) * go_ref[...] + bo_ref[...]
    xo_ref[...] = jnp.maximum(z, 0.0)
    p_ref[...] = jnp.sum(x, axis=0, keepdims=True)


_mlp_call = pl.pallas_call(
    _mlp_body,
    out_shape=(jax.ShapeDtypeStruct((N, H), jnp.float32),
               jax.ShapeDtypeStruct((1, H), jnp.float32)),
)


def _head_body(x4_ref, pstk_ref, wp_ref, bp_ref, out_ref):
    p4 = jnp.sum(x4_ref[...], axis=0, keepdims=True)
    acc = jnp.sum(bp_ref[...], axis=0, keepdims=True)
    for i in range(NLAYERS):
        acc = acc + jnp.dot(pstk_ref[i, :][None, :], wp_ref[i],
                            preferred_element_type=jnp.float32)
    acc = acc + jnp.dot(p4, wp_ref[NLAYERS], preferred_element_type=jnp.float32)
    out_ref[...] = acc


_head_call = pl.pallas_call(
    _head_body,
    out_shape=jax.ShapeDtypeStruct((1, O), jnp.float32),
)


def kernel(edge_index, h, embed, W1, W2, g_mlp, b_mlp, g_out, b_out, Wp, bp):
    u, v = edge_index[0], edge_index[1]
    src = jnp.concatenate([u, v])
    dst = jnp.concatenate([v, u])
    cap = 16 * NB * EB
    pad = cap - src.shape[0]
    src = jnp.concatenate([src, jnp.zeros((pad,), jnp.int32)])
    dst = jnp.concatenate([dst, jnp.full((pad,), N, jnp.int32)])
    srcb = src.reshape(16, NB, EB)
    srcb = jnp.concatenate(
        [srcb, jnp.zeros((16, NBP - NB, EB), jnp.int32)], axis=1)
    d0 = jnp.where(dst < HALF, dst, TRASH)
    d1 = jnp.where((dst >= HALF) & (dst < N), dst - HALF, TRASH)
    dstb = jnp.stack([d0.reshape(16, NB, EB), d1.reshape(16, NB, EB)])
    dstb = jnp.concatenate(
        [dstb, jnp.full((2, 16, NBP - NB, EB), TRASH, jnp.int32)], axis=2)
    hb = jnp.concatenate(
        [h, jnp.zeros((NE_PAD - N,), jnp.int32)]).reshape(32, EMB_B, EB)
    zeros_half = jnp.zeros((ROWS_SC, H), jnp.float32)

    x = _embed_call(embed, hb)
    pooled = []
    for i in range(NLAYERS):
        agg2 = _agg_call(x, srcb, dstb, zeros_half)
        x, p = _mlp_call(x, agg2, W1[i], W2[i],
                         g_mlp[i].reshape(1, H), b_mlp[i].reshape(1, H),
                         g_out[i].reshape(1, H), b_out[i].reshape(1, H))
        pooled.append(p)
    pstk = jnp.concatenate(pooled, axis=0)
    return _head_call(x, pstk, Wp, bp)


# speedup vs baseline: 1.0438x; 1.0438x over previous
import functools

import jax
import jax.numpy as jnp
from jax import lax
from jax.experimental import pallas as pl
from jax.experimental.pallas import tpu as pltpu
from jax.experimental.pallas import tpu_sc as plsc

N = 10000
H = 256
O = 128
NLAYERS = 4
HALF = 5000
ROWS_T = 5008
TRASH = 5000
EBK = 64
CHUNK_E = 2048
NCHK = 157
EMB_B = 3
EMB_EB = 128
NE_PAD = 32 * EMB_B * EMB_EB


@functools.cache
def _mesh():
    return plsc.VectorSubcoreMesh(core_axis_name="c", subcore_axis_name="s")


def _embed_body(embed_hbm, hb_hbm, out_hbm, idx_v, rows_v, sem):
    c = lax.axis_index("c")
    s = lax.axis_index("s")
    wid = s * 2 + c
    pltpu.sync_copy(hb_hbm.at[wid], idx_v)
    for j in range(EMB_B):
        pltpu.async_copy(embed_hbm.at[idx_v.at[j]], rows_v, sem).wait()
        pltpu.sync_copy(
            rows_v,
            out_hbm.at[pl.ds(wid * (EMB_B * EMB_EB) + j * EMB_EB, EMB_EB)])


@functools.cache
def _embed_call():
    return pl.kernel(
        _embed_body,
        mesh=_mesh(),
        out_type=jax.ShapeDtypeStruct((NE_PAD, H), jnp.float32),
        scratch_types=[
            pltpu.VMEM((EMB_B, EMB_EB), jnp.int32),
            pltpu.VMEM((EMB_EB, H), jnp.float32),
            pltpu.SemaphoreType.DMA,
        ],
    )


def _agg_body(xg_hbm, gsrc_hbm, dstl_hbm, out_hbm,
              gsrc_v, dstl_v, rows0, rows1, accum,
              sem0, sem1, sem_is, sem_id):
    co = lax.axis_index("c")
    s = lax.axis_index("s")
    iota = lax.iota(jnp.int32, 16)
    zeros16 = jnp.zeros((16,), jnp.float32)

    def zero(i, carry):
        for k in range(8):
            accum[pl.ds((i * 8 + k) * 16, 16)] = zeros16
        return carry

    lax.fori_loop(0, ROWS_T * 16 // 128, zero, 0)

    pltpu.async_copy(gsrc_hbm.at[0], gsrc_v.at[0], sem_is)
    pltpu.async_copy(dstl_hbm.at[co, 0], dstl_v.at[0], sem_id)

    def chunk(ci, carry):
        par = lax.rem(ci, 2)
        pltpu.make_async_copy(gsrc_hbm.at[0], gsrc_v.at[par], sem_is).wait()
        pltpu.make_async_copy(dstl_hbm.at[co, 0], dstl_v.at[par], sem_id).wait()

        def fix(i, c2):
            g = gsrc_v[par, pl.ds(i * 16, 16)]
            gsrc_v[par, pl.ds(i * 16, 16)] = g + s
            return c2

        lax.fori_loop(0, CHUNK_E // 16, fix, 0)

        pltpu.async_copy(xg_hbm.at[gsrc_v.at[par, pl.ds(0, EBK)]], rows0, sem0)
        pltpu.async_copy(xg_hbm.at[gsrc_v.at[par, pl.ds(EBK, EBK)]], rows1, sem1)
        pltpu.async_copy(gsrc_hbm.at[ci + 1], gsrc_v.at[1 - par], sem_is)
        pltpu.async_copy(dstl_hbm.at[co, ci + 1], dstl_v.at[1 - par], sem_id)

        parv = jnp.full((16,), par, jnp.int32)

        def pair(k, c3):
            for half, buf, sem in ((0, rows0, sem0), (1, rows1, sem1)):
                b = 2 * k + half
                base = b * EBK
                pltpu.make_async_copy(
                    xg_hbm.at[gsrc_v.at[par, pl.ds(0, EBK)]], buf, sem).wait()
                for e in range(EBK):
                    bc = plsc.load_gather(
                        dstl_v, [parv, jnp.full((16,), base + e, jnp.int32)])
                    plsc.addupdate_scatter(accum, [bc + iota], buf[e, :])

                @pl.when(k < (CHUNK_E // EBK // 2) - 1)
                def _():
                    pltpu.async_copy(
                        xg_hbm.at[gsrc_v.at[par, pl.ds((b + 2) * EBK, EBK)]],
                        buf, sem)
            return c3

        lax.fori_loop(0, CHUNK_E // EBK // 2, pair, 0)
        return carry

    lax.fori_loop(0, NCHK, chunk, 0)
    pltpu.make_async_copy(gsrc_hbm.at[0], gsrc_v.at[0], sem_is).wait()
    pltpu.make_async_copy(dstl_hbm.at[co, 0], dstl_v.at[0], sem_id).wait()
    pltpu.sync_copy(accum, out_hbm.at[co, s])


@functools.cache
def _agg_call():
    return pl.kernel(
        _agg_body,
        mesh=_mesh(),
        out_type=jax.ShapeDtypeStruct((2, 16, ROWS_T * 16), jnp.float32),
        scratch_types=[
            pltpu.VMEM((2, CHUNK_E), jnp.int32),
            pltpu.VMEM((2, CHUNK_E), jnp.int32),
            pltpu.VMEM((EBK, 16), jnp.float32),
            pltpu.VMEM((EBK, 16), jnp.float32),
            pltpu.VMEM((ROWS_T * 16,), jnp.float32),
            pltpu.SemaphoreType.DMA,
            pltpu.SemaphoreType.DMA,
            pltpu.SemaphoreType.DMA,
            pltpu.SemaphoreType.DMA,
        ],
        compiler_params=pltpu.CompilerParams(
            needs_layout_passes=False,
            use_tc_tiling_on_sc=False),
    )


def _mlp_body(x_ref, agg_ref, w1_ref, w2_ref, gm_ref, bm_ref, go_ref, bo_ref,
              xo_ref, p_ref):
    x = x_ref[:N, :]
    agg = agg_ref[...]
    z = 2.0 * x + agg
    z = jnp.dot(z, w1_ref[...], preferred_element_type=jnp.float32)
    m = jnp.mean(z, axis=0, keepdims=True)
    d = z - m
    var = jnp.mean(d * d, axis=0, keepdims=True)
    z = d * lax.rsqrt(var + 1e-5) * gm_ref[...] + bm_ref[...]
    z = jnp.maximum(z, 0.0)
    z = jnp.dot(z, w2_ref[...], preferred_element_type=jnp.float32)
    m2 = jnp.mean(z, axis=0, keepdims=True)
    d2 = z - m2
    var2 = jnp.mean(d2 * d2, axis=0, keepdims=True)
    z = d2 * lax.rsqrt(var2 + 1e-5) * go_ref[...] + bo_ref[...]
    xo_ref[...] = jnp.maximum(z, 0.0)
    p_ref[...] = jnp.sum(x, axis=0, keepdims=True)


_mlp_call = pl.pallas_call(
    _mlp_body,
    out_shape=(jax.ShapeDtypeStruct((N, H), jnp.float32),
               jax.ShapeDtypeStruct((1, H), jnp.float32)),
)


def _head_body(x4_ref, pstk_ref, wp_ref, bp_ref, out_ref):
    p4 = jnp.sum(x4_ref[...], axis=0, keepdims=True)
    acc = jnp.sum(bp_ref[...], axis=0, keepdims=True)
    for i in range(NLAYERS):
        acc = acc + jnp.dot(pstk_ref[i, :][None, :], wp_ref[i],
                            preferred_element_type=jnp.float32)
    acc = acc + jnp.dot(p4, wp_ref[NLAYERS], preferred_element_type=jnp.float32)
    out_ref[...] = acc


_head_call = pl.pallas_call(
    _head_body,
    out_shape=jax.ShapeDtypeStruct((1, O), jnp.float32),
)


def kernel(edge_index, h, embed, W1, W2, g_mlp, b_mlp, g_out, b_out, Wp, bp):
    u, v = edge_index[0], edge_index[1]
    src = jnp.concatenate([u, v])
    dst = jnp.concatenate([v, u])
    cap = NCHK * CHUNK_E
    pad = cap - src.shape[0]
    src = jnp.concatenate([src, jnp.arange(pad, dtype=jnp.int32) % N])
    dst = jnp.concatenate([dst, jnp.full((pad,), N, jnp.int32)])
    gsrc = (src * 16).reshape(NCHK, CHUNK_E)
    gsrc = jnp.concatenate([gsrc, jnp.zeros((1, CHUNK_E), jnp.int32)])
    d0 = jnp.where(dst < HALF, dst, TRASH) * 16
    d1 = jnp.where((dst >= HALF) & (dst < N), dst - HALF, TRASH) * 16
    dstl = jnp.stack([d0.reshape(NCHK, CHUNK_E), d1.reshape(NCHK, CHUNK_E)])
    dstl = jnp.concatenate(
        [dstl, jnp.full((2, 1, CHUNK_E), TRASH * 16, jnp.int32)], axis=1)
    hb = jnp.concatenate(
        [h, jnp.zeros((NE_PAD - N,), jnp.int32)]).reshape(32, EMB_B, EMB_EB)

    x = _embed_call()(embed, hb)
    pooled = []
    for i in range(NLAYERS):
        nx = x.shape[0]
        xg = x.reshape(nx * 16, 16)
        raw = _agg_call()(xg, gsrc, dstl)
        agg = (raw.reshape(2, 16, ROWS_T, 16)[:, :, :HALF, :]
               .transpose(0, 2, 1, 3).reshape(N, H))
        x, p = _mlp_call(x, agg, W1[i], W2[i],
                         g_mlp[i].reshape(1, H), b_mlp[i].reshape(1, H),
                         g_out[i].reshape(1, H), b_out[i].reshape(1, H))
        pooled.append(p)
    pstk = jnp.concatenate(pooled, axis=0)
    return _head_call(x, pstk, Wp, bp)


# speedup vs baseline: 1.7959x; 1.7204x over previous
import functools

import jax
import jax.numpy as jnp
from jax import lax
from jax.experimental import pallas as pl
from jax.experimental.pallas import tpu as pltpu
from jax.experimental.pallas import tpu_sc as plsc

N = 10000
H = 256
O = 128
NLAYERS = 4
HALF = 5000
ROWS_T = 5008
TRASH = 5000
EBK = 128
CHUNK_E = 1024
NCHK = 313
EMB_B = 3
EMB_EB = 128
NE_PAD = 32 * EMB_B * EMB_EB


@functools.cache
def _mesh():
    return plsc.VectorSubcoreMesh(core_axis_name="c", subcore_axis_name="s")


def _embed_body(embed_hbm, hb_hbm, out_hbm, idx_v, rows_v, sem):
    c = lax.axis_index("c")
    s = lax.axis_index("s")
    wid = s * 2 + c
    pltpu.sync_copy(hb_hbm.at[wid], idx_v)
    for j in range(EMB_B):
        pltpu.async_copy(embed_hbm.at[idx_v.at[j]], rows_v, sem).wait()
        pltpu.sync_copy(
            rows_v,
            out_hbm.at[pl.ds(wid * (EMB_B * EMB_EB) + j * EMB_EB, EMB_EB)])


@functools.cache
def _embed_call():
    return pl.kernel(
        _embed_body,
        mesh=_mesh(),
        out_type=jax.ShapeDtypeStruct((NE_PAD, H), jnp.float32),
        scratch_types=[
            pltpu.VMEM((EMB_B, EMB_EB), jnp.int32),
            pltpu.VMEM((EMB_EB, H), jnp.float32),
            pltpu.SemaphoreType.DMA,
        ],
    )


def _agg_body(xg_hbm, gsrc_hbm, idxe_hbm, out_hbm,
              gsrc_v, idxe_v, rows0, rows1, accum,
              sem0, sem1, sem_is, sem_id):
    co = lax.axis_index("c")
    s = lax.axis_index("s")
    zeros16 = jnp.zeros((16,), jnp.float32)

    @plsc.parallel_loop(0, ROWS_T * 16 // 16, unroll=8)
    def _zero(i):
        accum[pl.ds(i * 16, 16)] = zeros16

    pltpu.async_copy(gsrc_hbm.at[0], gsrc_v.at[0], sem_is)
    pltpu.async_copy(idxe_hbm.at[co, 0], idxe_v.at[0], sem_id)

    def chunk(ci, carry):
        par = lax.rem(ci, 2)
        pltpu.make_async_copy(gsrc_hbm.at[0], gsrc_v.at[par], sem_is).wait()
        pltpu.make_async_copy(idxe_hbm.at[co, 0], idxe_v.at[par], sem_id).wait()

        @plsc.parallel_loop(0, CHUNK_E // 16, unroll=4)
        def _fix(i):
            g = gsrc_v[par, pl.ds(i * 16, 16)]
            gsrc_v[par, pl.ds(i * 16, 16)] = g + s

        pltpu.async_copy(xg_hbm.at[gsrc_v.at[par, pl.ds(0, EBK)]], rows0, sem0)
        pltpu.async_copy(xg_hbm.at[gsrc_v.at[par, pl.ds(EBK, EBK)]], rows1, sem1)
        pltpu.async_copy(gsrc_hbm.at[ci + 1], gsrc_v.at[1 - par], sem_is)
        pltpu.async_copy(idxe_hbm.at[co, ci + 1], idxe_v.at[1 - par], sem_id)

        def pair(k, c3):
            for half, buf, sem in ((0, rows0, sem0), (1, rows1, sem1)):
                b = 2 * k + half
                base = b * EBK
                pltpu.make_async_copy(
                    xg_hbm.at[gsrc_v.at[par, pl.ds(0, EBK)]], buf, sem).wait()

                @plsc.parallel_loop(0, EBK, unroll=8)
                def _edges(e):
                    idx = idxe_v[par, base + e, :]
                    plsc.addupdate_scatter(accum, [idx], buf[e, :])

                @pl.when(k < (CHUNK_E // EBK // 2) - 1)
                def _():
                    pltpu.async_copy(
                        xg_hbm.at[gsrc_v.at[par, pl.ds((b + 2) * EBK, EBK)]],
                        buf, sem)
            return c3

        lax.fori_loop(0, CHUNK_E // EBK // 2, pair, 0)
        return carry

    lax.fori_loop(0, NCHK, chunk, 0)
    pltpu.make_async_copy(gsrc_hbm.at[0], gsrc_v.at[0], sem_is).wait()
    pltpu.make_async_copy(idxe_hbm.at[co, 0], idxe_v.at[0], sem_id).wait()
    pltpu.sync_copy(accum, out_hbm.at[co, s])


@functools.cache
def _agg_call():
    return pl.kernel(
        _agg_body,
        mesh=_mesh(),
        out_type=jax.ShapeDtypeStruct((2, 16, ROWS_T * 16), jnp.float32),
        scratch_types=[
            pltpu.VMEM((2, CHUNK_E), jnp.int32),
            pltpu.VMEM((2, CHUNK_E, 16), jnp.int32),
            pltpu.VMEM((EBK, 16), jnp.float32),
            pltpu.VMEM((EBK, 16), jnp.float32),
            pltpu.VMEM((ROWS_T * 16,), jnp.float32),
            pltpu.SemaphoreType.DMA,
            pltpu.SemaphoreType.DMA,
            pltpu.SemaphoreType.DMA,
            pltpu.SemaphoreType.DMA,
        ],
        compiler_params=pltpu.CompilerParams(
            needs_layout_passes=False,
            use_tc_tiling_on_sc=False),
    )


def _mlp_body(x_ref, agg_ref, w1_ref, w2_ref, gm_ref, bm_ref, go_ref, bo_ref,
              xo_ref, p_ref):
    x = x_ref[:N, :]
    agg = agg_ref[...]
    z = 2.0 * x + agg
    z = jnp.dot(z, w1_ref[...], preferred_element_type=jnp.float32)
    m = jnp.mean(z, axis=0, keepdims=True)
    d = z - m
    var = jnp.mean(d * d, axis=0, keepdims=True)
    z = d * lax.rsqrt(var + 1e-5) * gm_ref[...] + bm_ref[...]
    z = jnp.maximum(z, 0.0)
    z = jnp.dot(z, w2_ref[...], preferred_element_type=jnp.float32)
    m2 = jnp.mean(z, axis=0, keepdims=True)
    d2 = z - m2
    var2 = jnp.mean(d2 * d2, axis=0, keepdims=True)
    z = d2 * lax.rsqrt(var2 + 1e-5) * go_ref[...] + bo_ref[...]
    xo_ref[...] = jnp.maximum(z, 0.0)
    p_ref[...] = jnp.sum(x, axis=0, keepdims=True)


_mlp_call = pl.pallas_call(
    _mlp_body,
    out_shape=(jax.ShapeDtypeStruct((N, H), jnp.float32),
               jax.ShapeDtypeStruct((1, H), jnp.float32)),
)


def _head_body(x4_ref, pstk_ref, wp_ref, bp_ref, out_ref):
    p4 = jnp.sum(x4_ref[...], axis=0, keepdims=True)
    acc = jnp.sum(bp_ref[...], axis=0, keepdims=True)
    for i in range(NLAYERS):
        acc = acc + jnp.dot(pstk_ref[i, :][None, :], wp_ref[i],
                            preferred_element_type=jnp.float32)
    acc = acc + jnp.dot(p4, wp_ref[NLAYERS], preferred_element_type=jnp.float32)
    out_ref[...] = acc


_head_call = pl.pallas_call(
    _head_body,
    out_shape=jax.ShapeDtypeStruct((1, O), jnp.float32),
)


def kernel(edge_index, h, embed, W1, W2, g_mlp, b_mlp, g_out, b_out, Wp, bp):
    u, v = edge_index[0], edge_index[1]
    src = jnp.concatenate([u, v])
    dst = jnp.concatenate([v, u])
    cap = NCHK * CHUNK_E
    pad = cap - src.shape[0]
    src = jnp.concatenate([src, jnp.arange(pad, dtype=jnp.int32) % N])
    dst = jnp.concatenate([dst, jnp.full((pad,), N, jnp.int32)])
    gsrc = (src * 16).reshape(NCHK, CHUNK_E)
    gsrc = jnp.concatenate([gsrc, jnp.zeros((1, CHUNK_E), jnp.int32)])
    d0 = jnp.where(dst < HALF, dst, TRASH) * 16
    d1 = jnp.where((dst >= HALF) & (dst < N), dst - HALF, TRASH) * 16
    lane = jnp.arange(16, dtype=jnp.int32)
    idxe = jnp.stack([d0, d1])[:, :, None] + lane
    idxe = idxe.reshape(2, NCHK, CHUNK_E, 16)
    idxe = jnp.concatenate(
        [idxe, jnp.full((2, 1, CHUNK_E, 16), TRASH * 16, jnp.int32)], axis=1)
    hb = jnp.concatenate(
        [h, jnp.zeros((NE_PAD - N,), jnp.int32)]).reshape(32, EMB_B, EMB_EB)

    x = _embed_call()(embed, hb)
    pooled = []
    for i in range(NLAYERS):
        nx = x.shape[0]
        xg = x.reshape(nx * 16, 16)
        raw = _agg_call()(xg, gsrc, idxe)
        agg = (raw.reshape(2, 16, ROWS_T, 16)[:, :, :HALF, :]
               .transpose(0, 2, 1, 3).reshape(N, H))
        x, p = _mlp_call(x, agg, W1[i], W2[i],
                         g_mlp[i].reshape(1, H), b_mlp[i].reshape(1, H),
                         g_out[i].reshape(1, H), b_out[i].reshape(1, H))
        pooled.append(p)
    pstk = jnp.concatenate(pooled, axis=0)
    return _head_call(x, pstk, Wp, bp)


# speedup vs baseline: 3.2709x; 1.8214x over previous
import functools

import jax
import jax.numpy as jnp
from jax import lax
from jax.experimental import pallas as pl
from jax.experimental.pallas import tpu as pltpu
from jax.experimental.pallas import tpu_sc as plsc

N = 10000
H = 256
O = 128
NLAYERS = 4
HALF = 5000
ROWS_T = 5008
TRASH = 5000
EBK = 128
CHUNK_E = 8192
NCHK = 40
EMB_B = 3
EMB_EB = 128
NE_PAD = 32 * EMB_B * EMB_EB


@functools.cache
def _mesh():
    return plsc.VectorSubcoreMesh(core_axis_name="c", subcore_axis_name="s")


def _embed_body(embed_hbm, hb_hbm, out_hbm, idx_v, rows_v, sem):
    c = lax.axis_index("c")
    s = lax.axis_index("s")
    wid = s * 2 + c
    pltpu.sync_copy(hb_hbm.at[wid], idx_v)
    for j in range(EMB_B):
        pltpu.async_copy(embed_hbm.at[idx_v.at[j]], rows_v, sem).wait()
        pltpu.sync_copy(
            rows_v,
            out_hbm.at[pl.ds(wid * (EMB_B * EMB_EB) + j * EMB_EB, EMB_EB)])


@functools.cache
def _embed_call():
    return pl.kernel(
        _embed_body,
        mesh=_mesh(),
        out_type=jax.ShapeDtypeStruct((NE_PAD, H), jnp.float32),
        scratch_types=[
            pltpu.VMEM((EMB_B, EMB_EB), jnp.int32),
            pltpu.VMEM((EMB_EB, H), jnp.float32),
            pltpu.SemaphoreType.DMA,
        ],
    )


def _agg_body(xg_hbm, gsrc_hbm, dstl_hbm, out_hbm,
              gsrc_v, dstl_v, rows0, rows1, rows2, rows3, accum,
              sem0, sem1, sem2, sem3, sem_is, sem_id):
    co = lax.axis_index("c")
    s = lax.axis_index("s")
    iota = lax.iota(jnp.int32, 16)
    zeros16 = jnp.zeros((16,), jnp.float32)
    rows = (rows0, rows1, rows2, rows3)
    sems = (sem0, sem1, sem2, sem3)
    nbuf = len(rows)
    nblk = CHUNK_E // EBK

    @plsc.parallel_loop(0, ROWS_T * 16 // 16, unroll=8)
    def _zero(i):
        accum[pl.ds(i * 16, 16)] = zeros16

    pltpu.async_copy(gsrc_hbm.at[0], gsrc_v.at[0], sem_is)
    pltpu.async_copy(dstl_hbm.at[co, 0], dstl_v.at[0], sem_id)

    def chunk(ci, carry):
        par = lax.rem(ci, 2)
        pltpu.make_async_copy(gsrc_hbm.at[0], gsrc_v.at[par], sem_is).wait()
        pltpu.make_async_copy(dstl_hbm.at[co, 0], dstl_v.at[par], sem_id).wait()

        @plsc.parallel_loop(0, CHUNK_E // 16, unroll=4)
        def _fix(i):
            g = gsrc_v[par, pl.ds(i * 16, 16)]
            gsrc_v[par, pl.ds(i * 16, 16)] = g + s

        for q in range(nbuf):
            pltpu.async_copy(
                xg_hbm.at[gsrc_v.at[par, pl.ds(q * EBK, EBK)]], rows[q], sems[q])
        pltpu.async_copy(gsrc_hbm.at[ci + 1], gsrc_v.at[1 - par], sem_is)
        pltpu.async_copy(dstl_hbm.at[co, ci + 1], dstl_v.at[1 - par], sem_id)

        parv = jnp.full((16,), par, jnp.int32)

        def quad(k, c3):
            for q in range(nbuf):
                b = nbuf * k + q
                base = b * EBK
                buf, sem = rows[q], sems[q]
                pltpu.make_async_copy(
                    xg_hbm.at[gsrc_v.at[par, pl.ds(0, EBK)]], buf, sem).wait()

                @plsc.parallel_loop(0, EBK, unroll=8)
                def _edges(e):
                    bc = plsc.load_gather(
                        dstl_v, [parv, jnp.full((16,), base + e, jnp.int32)])
                    plsc.addupdate_scatter(accum, [bc + iota], buf[e, :])

                @pl.when(k < nblk // nbuf - 1)
                def _():
                    pltpu.async_copy(
                        xg_hbm.at[gsrc_v.at[par, pl.ds((b + nbuf) * EBK, EBK)]],
                        buf, sem)
            return c3

        lax.fori_loop(0, nblk // nbuf, quad, 0)
        return carry

    lax.fori_loop(0, NCHK, chunk, 0)
    pltpu.make_async_copy(gsrc_hbm.at[0], gsrc_v.at[0], sem_is).wait()
    pltpu.make_async_copy(dstl_hbm.at[co, 0], dstl_v.at[0], sem_id).wait()
    pltpu.sync_copy(accum, out_hbm.at[co, s])


@functools.cache
def _agg_call():
    return pl.kernel(
        _agg_body,
        mesh=_mesh(),
        out_type=jax.ShapeDtypeStruct((2, 16, ROWS_T * 16), jnp.float32),
        scratch_types=[
            pltpu.VMEM((2, CHUNK_E), jnp.int32),
            pltpu.VMEM((2, CHUNK_E), jnp.int32),
            pltpu.VMEM((EBK, 16), jnp.float32),
            pltpu.VMEM((EBK, 16), jnp.float32),
            pltpu.VMEM((EBK, 16), jnp.float32),
            pltpu.VMEM((EBK, 16), jnp.float32),
            pltpu.VMEM((ROWS_T * 16,), jnp.float32),
            pltpu.SemaphoreType.DMA,
            pltpu.SemaphoreType.DMA,
            pltpu.SemaphoreType.DMA,
            pltpu.SemaphoreType.DMA,
            pltpu.SemaphoreType.DMA,
            pltpu.SemaphoreType.DMA,
        ],
        compiler_params=pltpu.CompilerParams(
            needs_layout_passes=False,
            use_tc_tiling_on_sc=False),
    )


def _mlp_body(x_ref, agg_ref, w1_ref, w2_ref, gm_ref, bm_ref, go_ref, bo_ref,
              xo_ref, p_ref):
    x = x_ref[:N, :]
    agg = agg_ref[...]
    z = 2.0 * x + agg
    z = jnp.dot(z, w1_ref[...], preferred_element_type=jnp.float32)
    m = jnp.mean(z, axis=0, keepdims=True)
    d = z - m
    var = jnp.mean(d * d, axis=0, keepdims=True)
    z = d * lax.rsqrt(var + 1e-5) * gm_ref[...] + bm_ref[...]
    z = jnp.maximum(z, 0.0)
    z = jnp.dot(z, w2_ref[...], preferred_element_type=jnp.float32)
    m2 = jnp.mean(z, axis=0, keepdims=True)
    d2 = z - m2
    var2 = jnp.mean(d2 * d2, axis=0, keepdims=True)
    z = d2 * lax.rsqrt(var2 + 1e-5) * go_ref[...] + bo_ref[...]
    xo_ref[...] = jnp.maximum(z, 0.0)
    p_ref[...] = jnp.sum(x, axis=0, keepdims=True)


_mlp_call = pl.pallas_call(
    _mlp_body,
    out_shape=(jax.ShapeDtypeStruct((N, H), jnp.float32),
               jax.ShapeDtypeStruct((1, H), jnp.float32)),
)


def _head_body(x4_ref, pstk_ref, wp_ref, bp_ref, out_ref):
    p4 = jnp.sum(x4_ref[...], axis=0, keepdims=True)
    acc = jnp.sum(bp_ref[...], axis=0, keepdims=True)
    for i in range(NLAYERS):
        acc = acc + jnp.dot(pstk_ref[i, :][None, :], wp_ref[i],
                            preferred_element_type=jnp.float32)
    acc = acc + jnp.dot(p4, wp_ref[NLAYERS], preferred_element_type=jnp.float32)
    out_ref[...] = acc


_head_call = pl.pallas_call(
    _head_body,
    out_shape=jax.ShapeDtypeStruct((1, O), jnp.float32),
)


def kernel(edge_index, h, embed, W1, W2, g_mlp, b_mlp, g_out, b_out, Wp, bp):
    u, v = edge_index[0], edge_index[1]
    src = jnp.concatenate([u, v])
    dst = jnp.concatenate([v, u])
    cap = NCHK * CHUNK_E
    pad = cap - src.shape[0]
    src = jnp.concatenate([src, jnp.arange(pad, dtype=jnp.int32) % N])
    dst = jnp.concatenate([dst, jnp.full((pad,), N, jnp.int32)])
    gsrc = (src * 16).reshape(NCHK, CHUNK_E)
    gsrc = jnp.concatenate([gsrc, jnp.zeros((1, CHUNK_E), jnp.int32)])
    d0 = jnp.where(dst < HALF, dst, TRASH) * 16
    d1 = jnp.where((dst >= HALF) & (dst < N), dst - HALF, TRASH) * 16
    dstl = jnp.stack([d0.reshape(NCHK, CHUNK_E), d1.reshape(NCHK, CHUNK_E)])
    dstl = jnp.concatenate(
        [dstl, jnp.full((2, 1, CHUNK_E), TRASH * 16, jnp.int32)], axis=1)
    hb = jnp.concatenate(
        [h, jnp.zeros((NE_PAD - N,), jnp.int32)]).reshape(32, EMB_B, EMB_EB)

    x = _embed_call()(embed, hb)
    pooled = []
    for i in range(NLAYERS):
        nx = x.shape[0]
        xg = x.reshape(nx * 16, 16)
        raw = _agg_call()(xg, gsrc, dstl)
        agg = (raw.reshape(2, 16, ROWS_T, 16)[:, :, :HALF, :]
               .transpose(0, 2, 1, 3).reshape(N, H))
        x, p = _mlp_call(x, agg, W1[i], W2[i],
                         g_mlp[i].reshape(1, H), b_mlp[i].reshape(1, H),
                         g_out[i].reshape(1, H), b_out[i].reshape(1, H))
        pooled.append(p)
    pstk = jnp.concatenate(pooled, axis=0)
    return _head_call(x, pstk, Wp, bp)


# speedup vs baseline: 3.8626x; 1.1809x over previous
import functools

import jax
import jax.numpy as jnp
from jax import lax
from jax.experimental import pallas as pl
from jax.experimental.pallas import tpu as pltpu
from jax.experimental.pallas import tpu_sc as plsc

N = 10000
H = 256
O = 128
NLAYERS = 4
HALF = 5000
ROWS_T = 5008
TRASH = 5000
EBK = 128
CHUNK_E = 8192
NCHK = 40
EMB_B = 3
EMB_EB = 128
NE_PAD = 32 * EMB_B * EMB_EB


@functools.cache
def _mesh():
    return plsc.VectorSubcoreMesh(core_axis_name="c", subcore_axis_name="s")


def _embed_body(embed_hbm, hb_hbm, out_hbm, idx_v, rows_v, sem):
    c = lax.axis_index("c")
    s = lax.axis_index("s")
    wid = s * 2 + c
    pltpu.sync_copy(hb_hbm.at[wid], idx_v)
    for j in range(EMB_B):
        pltpu.async_copy(embed_hbm.at[idx_v.at[j]], rows_v, sem).wait()
        pltpu.sync_copy(
            rows_v,
            out_hbm.at[pl.ds(wid * (EMB_B * EMB_EB) + j * EMB_EB, EMB_EB)])


@functools.cache
def _embed_call():
    return pl.kernel(
        _embed_body,
        mesh=_mesh(),
        out_type=jax.ShapeDtypeStruct((NE_PAD, H), jnp.float32),
        scratch_types=[
            pltpu.VMEM((EMB_B, EMB_EB), jnp.int32),
            pltpu.VMEM((EMB_EB, H), jnp.float32),
            pltpu.SemaphoreType.DMA,
        ],
    )


def _agg_body(xg_hbm, gsrc_hbm, dstl_hbm, out_hbm,
              gsrc_v, dstl_v, rows0, rows1, rows2, rows3, accum,
              sem0, sem1, sem2, sem3, sem_is, sem_id):
    co = lax.axis_index("c")
    s = lax.axis_index("s")
    iota = lax.iota(jnp.int32, 16)
    zeros16 = jnp.zeros((16,), jnp.float32)
    rows = (rows0, rows1, rows2, rows3)
    sems = (sem0, sem1, sem2, sem3)
    nbuf = len(rows)
    nblk = CHUNK_E // EBK

    @plsc.parallel_loop(0, ROWS_T, unroll=8)
    def _zero(i):
        accum[i, :] = zeros16

    pltpu.async_copy(gsrc_hbm.at[0], gsrc_v.at[0], sem_is)
    pltpu.async_copy(dstl_hbm.at[co, 0], dstl_v.at[0], sem_id)

    def chunk(ci, carry):
        par = lax.rem(ci, 2)
        pltpu.make_async_copy(gsrc_hbm.at[0], gsrc_v.at[par], sem_is).wait()
        pltpu.make_async_copy(dstl_hbm.at[co, 0], dstl_v.at[par], sem_id).wait()

        @plsc.parallel_loop(0, CHUNK_E // 16, unroll=4)
        def _fix(i):
            g = gsrc_v[par, pl.ds(i * 16, 16)]
            gsrc_v[par, pl.ds(i * 16, 16)] = g + s

        for q in range(nbuf):
            pltpu.async_copy(
                xg_hbm.at[gsrc_v.at[par, pl.ds(q * EBK, EBK)]], rows[q], sems[q])
        pltpu.async_copy(gsrc_hbm.at[ci + 1], gsrc_v.at[1 - par], sem_is)
        pltpu.async_copy(dstl_hbm.at[co, ci + 1], dstl_v.at[1 - par], sem_id)

        parv = jnp.full((16,), par, jnp.int32)

        def quad(k, c3):
            for q in range(nbuf):
                b = nbuf * k + q
                base = b * EBK
                buf, sem = rows[q], sems[q]
                pltpu.make_async_copy(
                    xg_hbm.at[gsrc_v.at[par, pl.ds(0, EBK)]], buf, sem).wait()

                @plsc.parallel_loop(0, EBK, unroll=8)
                def _edges(e):
                    bc = plsc.load_gather(
                        dstl_v, [parv, jnp.full((16,), base + e, jnp.int32)])
                    plsc.addupdate_scatter(accum, [bc, iota], buf[e, :])

                @pl.when(k < nblk // nbuf - 1)
                def _():
                    pltpu.async_copy(
                        xg_hbm.at[gsrc_v.at[par, pl.ds((b + nbuf) * EBK, EBK)]],
                        buf, sem)
            return c3

        lax.fori_loop(0, nblk // nbuf, quad, 0)
        return carry

    lax.fori_loop(0, NCHK, chunk, 0)
    pltpu.make_async_copy(gsrc_hbm.at[0], gsrc_v.at[0], sem_is).wait()
    pltpu.make_async_copy(dstl_hbm.at[co, 0], dstl_v.at[0], sem_id).wait()
    pltpu.sync_copy(accum.at[pl.ds(0, HALF)],
                    out_hbm.at[pl.ds(co * HALF, HALF), pl.ds(s * 16, 16)])


@functools.cache
def _agg_call():
    return pl.kernel(
        _agg_body,
        mesh=_mesh(),
        out_type=jax.ShapeDtypeStruct((N, H), jnp.float32),
        scratch_types=[
            pltpu.VMEM((2, CHUNK_E), jnp.int32),
            pltpu.VMEM((2, CHUNK_E), jnp.int32),
            pltpu.VMEM((EBK, 16), jnp.float32),
            pltpu.VMEM((EBK, 16), jnp.float32),
            pltpu.VMEM((EBK, 16), jnp.float32),
            pltpu.VMEM((EBK, 16), jnp.float32),
            pltpu.VMEM((ROWS_T, 16), jnp.float32),
            pltpu.SemaphoreType.DMA,
            pltpu.SemaphoreType.DMA,
            pltpu.SemaphoreType.DMA,
            pltpu.SemaphoreType.DMA,
            pltpu.SemaphoreType.DMA,
            pltpu.SemaphoreType.DMA,
        ],
        compiler_params=pltpu.CompilerParams(
            needs_layout_passes=False,
            use_tc_tiling_on_sc=False),
    )


def _mlp_body(x_ref, agg_ref, w1_ref, w2_ref, gm_ref, bm_ref, go_ref, bo_ref,
              xo_ref, p_ref):
    x = x_ref[:N, :]
    agg = agg_ref[...]
    z = 2.0 * x + agg
    z = jnp.dot(z, w1_ref[...], preferred_element_type=jnp.float32)
    m = jnp.mean(z, axis=0, keepdims=True)
    d = z - m
    var = jnp.mean(d * d, axis=0, keepdims=True)
    z = d * lax.rsqrt(var + 1e-5) * gm_ref[...] + bm_ref[...]
    z = jnp.maximum(z, 0.0)
    z = jnp.dot(z, w2_ref[...], preferred_element_type=jnp.float32)
    m2 = jnp.mean(z, axis=0, keepdims=True)
    d2 = z - m2
    var2 = jnp.mean(d2 * d2, axis=0, keepdims=True)
    z = d2 * lax.rsqrt(var2 + 1e-5) * go_ref[...] + bo_ref[...]
    xo_ref[...] = jnp.maximum(z, 0.0)
    p_ref[...] = jnp.sum(x, axis=0, keepdims=True)


_mlp_call = pl.pallas_call(
    _mlp_body,
    out_shape=(jax.ShapeDtypeStruct((N, H), jnp.float32),
               jax.ShapeDtypeStruct((1, H), jnp.float32)),
)


def _head_body(x4_ref, pstk_ref, wp_ref, bp_ref, out_ref):
    p4 = jnp.sum(x4_ref[...], axis=0, keepdims=True)
    acc = jnp.sum(bp_ref[...], axis=0, keepdims=True)
    for i in range(NLAYERS):
        acc = acc + jnp.dot(pstk_ref[i, :][None, :], wp_ref[i],
                            preferred_element_type=jnp.float32)
    acc = acc + jnp.dot(p4, wp_ref[NLAYERS], preferred_element_type=jnp.float32)
    out_ref[...] = acc


_head_call = pl.pallas_call(
    _head_body,
    out_shape=jax.ShapeDtypeStruct((1, O), jnp.float32),
)


def kernel(edge_index, h, embed, W1, W2, g_mlp, b_mlp, g_out, b_out, Wp, bp):
    u, v = edge_index[0], edge_index[1]
    src = jnp.concatenate([u, v])
    dst = jnp.concatenate([v, u])
    cap = NCHK * CHUNK_E
    pad = cap - src.shape[0]
    src = jnp.concatenate([src, jnp.arange(pad, dtype=jnp.int32) % N])
    dst = jnp.concatenate([dst, jnp.full((pad,), N, jnp.int32)])
    gsrc = (src * 16).reshape(NCHK, CHUNK_E)
    gsrc = jnp.concatenate([gsrc, jnp.zeros((1, CHUNK_E), jnp.int32)])
    d0 = jnp.where(dst < HALF, dst, TRASH)
    d1 = jnp.where((dst >= HALF) & (dst < N), dst - HALF, TRASH)
    dstl = jnp.stack([d0.reshape(NCHK, CHUNK_E), d1.reshape(NCHK, CHUNK_E)])
    dstl = jnp.concatenate(
        [dstl, jnp.full((2, 1, CHUNK_E), TRASH, jnp.int32)], axis=1)
    hb = jnp.concatenate(
        [h, jnp.zeros((NE_PAD - N,), jnp.int32)]).reshape(32, EMB_B, EMB_EB)

    x = _embed_call()(embed, hb)
    pooled = []
    for i in range(NLAYERS):
        nx = x.shape[0]
        xg = x.reshape(nx * 16, 16)
        agg = _agg_call()(xg, gsrc, dstl)
        x, p = _mlp_call(x, agg, W1[i], W2[i],
                         g_mlp[i].reshape(1, H), b_mlp[i].reshape(1, H),
                         g_out[i].reshape(1, H), b_out[i].reshape(1, H))
        pooled.append(p)
    pstk = jnp.concatenate(pooled, axis=0)
    return _head_call(x, pstk, Wp, bp)
